# Initial kernel scaffold; baseline (speedup 1.0000x reference)
import jax
import jax.numpy as jnp
from jax.experimental import pallas as pl


def kernel(x, edge_index, batch, W_gat, att_src, att_dst, b_gat, W_gcn, b_gcn, W1, b1, W2, b2, W3, b3, W4, b4, W5, b5):
    raise NotImplementedError("write your pallas kernel here")


# speedup vs baseline: 4.7178x; 4.7178x over previous
import functools
import jax
import jax.numpy as jnp
import numpy as np
from jax import lax
from jax.experimental import pallas as pl
from jax.experimental.pallas import tpu as pltpu

N = 10000
E = 160000
F = 66
HEADS = 10
HOUT = 66
D_GAT = HEADS * HOUT
D_GCN = D_GAT * 2
DP = 672
E2 = E + N
EP = 170240


def _leaky(v, slope):
    return jnp.where(v >= 0, v, slope * v)


def _tc1_body(x_ref, wg_ref, asrc_ref, adst_ref, h_ref, as_ref, ad_ref):
    h = jnp.dot(x_ref[...], wg_ref[...], preferred_element_type=jnp.float32)
    h_ref[...] = h
    as_ref[...] = jnp.dot(h, asrc_ref[...], preferred_element_type=jnp.float32)
    ad_ref[...] = jnp.dot(h, adst_ref[...], preferred_element_type=jnp.float32)


def _tc1(x, wg_pad, A_src, A_dst):
    R = 1000
    grid = (N // R,)
    return pl.pallas_call(
        _tc1_body,
        grid=grid,
        in_specs=[
            pl.BlockSpec((R, F), lambda i: (i, 0)),
            pl.BlockSpec((F, DP), lambda i: (0, 0)),
            pl.BlockSpec((DP, 16), lambda i: (0, 0)),
            pl.BlockSpec((DP, 16), lambda i: (0, 0)),
        ],
        out_specs=[
            pl.BlockSpec((R, DP), lambda i: (i, 0)),
            pl.BlockSpec((R, 16), lambda i: (i, 0)),
            pl.BlockSpec((R, 16), lambda i: (i, 0)),
        ],
        out_shape=[
            jax.ShapeDtypeStruct((N, DP), jnp.float32),
            jax.ShapeDtypeStruct((N, 16), jnp.float32),
            jax.ShapeDtypeStruct((N, 16), jnp.float32),
        ],
    )(x, wg_pad, A_src, A_dst)


def _tc2_body(den_ref, agg_ref, P_ref, bg_ref, x1t_ref, dinv_ref):
    den = den_ref[...]
    agg = agg_ref[...]
    deg = den[:, 10:11]
    dinv = jnp.where(deg > 0, lax.rsqrt(deg), 0.0)
    inv_den = 1.0 / (den + 1e-16)
    invexp = jnp.dot(inv_den, P_ref[...], preferred_element_type=jnp.float32)
    x1 = _leaky(agg * invexp + bg_ref[...], 0.01)
    x1t_ref[...] = x1 * dinv
    dinv_ref[...] = dinv


def _tc2(den, agg, P, bg_pad):
    R = 1000
    grid = (N // R,)
    return pl.pallas_call(
        _tc2_body,
        grid=grid,
        in_specs=[
            pl.BlockSpec((R, 16), lambda i: (i, 0)),
            pl.BlockSpec((R, DP), lambda i: (i, 0)),
            pl.BlockSpec((16, DP), lambda i: (0, 0)),
            pl.BlockSpec((1, DP), lambda i: (0, 0)),
        ],
        out_specs=[
            pl.BlockSpec((R, DP), lambda i: (i, 0)),
            pl.BlockSpec((R, 1), lambda i: (i, 0)),
        ],
        out_shape=[
            jax.ShapeDtypeStruct((N, DP), jnp.float32),
            jax.ShapeDtypeStruct((N, 1), jnp.float32),
        ],
    )(den, agg, P, bg_pad)


def _tc3_body(agg2_ref, dinv_ref, wgcn_ref, bgcn_ref, w1_ref, b1_ref,
              w2_ref, b2_ref, w3_ref, b3_ref, w4_ref, b4_ref, w5_ref, b5_ref,
              out_ref):
    a = agg2_ref[...] * dinv_ref[...]
    x2 = _leaky(jnp.dot(a, wgcn_ref[...], preferred_element_type=jnp.float32)
                + bgcn_ref[...], 0.01)
    x3 = _leaky(jnp.dot(x2, w1_ref[...], preferred_element_type=jnp.float32)
                + b1_ref[...], 0.01)
    x4 = _leaky(jnp.dot(x3, w2_ref[...], preferred_element_type=jnp.float32)
                + b2_ref[...], 0.01)
    x5 = _leaky(jnp.dot(x4, w3_ref[...], preferred_element_type=jnp.float32)
                + b3_ref[...], 0.01)
    x6 = _leaky(jnp.dot(x5, w4_ref[...], preferred_element_type=jnp.float32)
                + b4_ref[...], 0.01)
    out_ref[...] = (jnp.dot(x6, w5_ref[...], preferred_element_type=jnp.float32)
                    + b5_ref[...])


def _tc3(agg2, dinv, wgcn_pad, bgcn, W1, b1, W2, b2, W3, b3, W4, b4, W5, b5):
    R = 1000
    grid = (N // R,)
    full = lambda r, c: pl.BlockSpec((r, c), lambda i: (0, 0))
    return pl.pallas_call(
        _tc3_body,
        grid=grid,
        in_specs=[
            pl.BlockSpec((R, DP), lambda i: (i, 0)),
            pl.BlockSpec((R, 1), lambda i: (i, 0)),
            full(DP, D_GCN), full(1, D_GCN),
            full(D_GCN, 1000), full(1, 1000),
            full(1000, 64), full(1, 64),
            full(64, 32), full(1, 32),
            full(32, 16), full(1, 16),
            full(16, 1), full(1, 1),
        ],
        out_specs=pl.BlockSpec((R, 1), lambda i: (i, 0)),
        out_shape=jax.ShapeDtypeStruct((N, 1), jnp.float32),
    )(agg2, dinv, wgcn_pad, bgcn.reshape(1, -1), W1, b1.reshape(1, -1),
      W2, b2.reshape(1, -1), W3, b3.reshape(1, -1), W4, b4.reshape(1, -1),
      W5, b5.reshape(1, -1))


def _edges_gat_jnp(src, dst, h, a_s, a_d):
    e = a_s[src] + a_d[dst]
    e = _leaky(e, 0.2)
    w = jnp.exp(e)
    valid = (dst < N)
    w = jnp.where(valid[:, None], w, 0.0)
    den = jax.ops.segment_sum(w, jnp.where(valid, dst, 0), num_segments=N)
    msg = h[src] * w[:, _PAT_ALL]
    agg = jax.ops.segment_sum(jnp.where(valid[:, None], msg, 0.0),
                              jnp.where(valid, dst, 0), num_segments=N)
    return den, agg


def _edges_gcn_jnp(src, dst, x1t):
    valid = (dst < N)
    g = jnp.where(valid[:, None], x1t[src], 0.0)
    return jax.ops.segment_sum(g, jnp.where(valid, dst, 0), num_segments=N)


_PAT_ALL = np.minimum(np.arange(DP) // HOUT, 15)


def kernel(x, edge_index, batch, W_gat, att_src, att_dst, b_gat, W_gcn, b_gcn,
           W1, b1, W2, b2, W3, b3, W4, b4, W5, b5):
    f32 = jnp.float32
    loop = jnp.arange(N, dtype=jnp.int32)
    padn = EP - E2
    src = jnp.concatenate([edge_index[0].astype(jnp.int32), loop,
                           jnp.zeros((padn,), jnp.int32)])
    dst = jnp.concatenate([edge_index[1].astype(jnp.int32), loop,
                           jnp.full((padn,), N, jnp.int32)])

    wg_pad = jnp.zeros((F, DP), f32).at[:, :D_GAT].set(W_gat)
    cols = np.arange(D_GAT)
    A_src = jnp.zeros((DP, 16), f32).at[cols, cols // HOUT].set(
        att_src.reshape(HEADS, HOUT)[cols // HOUT, cols % HOUT])
    A_dst = jnp.zeros((DP, 16), f32).at[cols, cols // HOUT].set(
        att_dst.reshape(HEADS, HOUT)[cols // HOUT, cols % HOUT])
    P = jnp.zeros((16, DP), f32).at[cols // HOUT, cols].set(1.0)
    bg_pad = jnp.zeros((1, DP), f32).at[0, :D_GAT].set(b_gat)
    wgcn_pad = jnp.zeros((DP, D_GCN), f32).at[:D_GAT, :].set(W_gcn)

    h, a_s, a_d = _tc1(x, wg_pad, A_src, A_dst)
    den, agg = _edges_gat_jnp(src, dst, h, a_s, a_d)
    x1t, dinv = _tc2(den, agg, P, bg_pad)
    agg2 = _edges_gcn_jnp(src, dst, x1t)
    return _tc3(agg2, dinv, wgcn_pad, b_gcn, W1, b1, W2, b2, W3, b3,
                W4, b4, W5, b5)
